# baseline (speedup 1.0000x reference)
import functools

import jax
import jax.numpy as jnp
from jax import lax
from jax.experimental import pallas as pl
from jax.experimental.pallas import tpu as pltpu
from jax.experimental.pallas import tpu_sc as plsc

ND = 13
NSP = 26
VOC = 1000000
BT = 16384
FLAT = 32000000

_info = plsc.get_sparse_core_info()
NCORE = _info.num_cores
NSUB = _info.num_subcores
NW = NCORE * NSUB
BPW = BT // NW
NCHUNK = BPW // 16
NIDX = NSP * BPW
GCH = 128
NDMA = NIDX // GCH

W = 2048
NFULL = 488
TAILC = NFULL * W
W2 = 512
RAGC = TAILC + W2
NUNIT = NFULL * 4
NJ = (NUNIT + NW - 1) // NW

_mesh = plsc.VectorSubcoreMesh(core_axis_name="c", subcore_axis_name="s")


@functools.partial(
    pl.kernel,
    mesh=_mesh,
    out_type=jax.ShapeDtypeStruct((FLAT,), jnp.float32),
    scratch_types=[
        pltpu.VMEM((2, 8, W), jnp.float32),
        pltpu.VMEM((8 * W,), jnp.float32),
        pltpu.VMEM((8 * W,), jnp.float32),
        pltpu.VMEM((8, W2), jnp.float32),
        pltpu.VMEM((8 * W2,), jnp.float32),
        pltpu.SemaphoreType.DMA,
        pltpu.SemaphoreType.DMA,
        pltpu.SemaphoreType.DMA,
    ],
)
def _stage(tab_hbm, tail_hbm, flat, blk, rowb0, rowb1, blk2, rowb2,
           insem0, insem1, outsem):
    wid = lax.axis_index("s") * NCORE + lax.axis_index("c")
    insems = (insem0, insem1)
    rowbs = (rowb0, rowb1)

    def unit(j):
        u = wid + NW * j
        rb = u // NFULL
        c0 = (u % NFULL) * W
        r0 = pl.multiple_of(rb * 8, 8)
        c0 = pl.multiple_of(c0, 128)
        ok = jnp.logical_and(u >= 0, u < NUNIT)
        return ok, rb, r0, c0

    def in_copy(j, slot, t):
        _, rb, r0, c0 = unit(j)
        ct = pl.multiple_of(c0 + t * 128, 128)
        return pltpu.make_async_copy(
            tab_hbm.at[pl.ds(r0, 8), pl.ds(ct, 128)],
            blk.at[slot, :, pl.ds(t * 128, 128)], insems[slot],
        )

    def fire_in(j, slot):
        ok = unit(j)[0]

        def _go():
            for t in range(W // 128):
                in_copy(j, slot, t).start()
        pl.when(ok)(_go)

    def wait_in(j, slot):
        ok = unit(j)[0]

        def _go():
            for t in range(W // 128):
                in_copy(j, slot, t).wait()
        pl.when(ok)(_go)

    def rearrange(j, slot):
        ok = unit(j)[0]

        def _go():
            def inner(g, carry):
                off = g * 16
                for r in range(8):
                    v = blk[slot, r, pl.ds(off, 16)]
                    rowbs[slot][pl.ds(r * W + off, 16)] = v
                return carry
            lax.fori_loop(0, W // 16, inner, 0, unroll=4)
        pl.when(ok)(_go)

    def out_copy(j, slot, r):
        _, rb, r0, c0 = unit(j)
        o = pl.multiple_of((rb * 8 + r) * VOC + c0, 8)
        return pltpu.make_async_copy(
            rowbs[slot].at[pl.ds(r * W, W)], flat.at[pl.ds(o, W)], outsem,
        )

    def fire_outs(j, slot):
        ok = unit(j)[0]

        def _go():
            for r in range(8):
                out_copy(j, slot, r).start()
        pl.when(ok)(_go)

    def drain_outs(j, slot):
        ok = unit(j)[0]

        def _go():
            for r in range(8):
                out_copy(j, slot, r).wait()
        pl.when(ok)(_go)

    fire_in(0, 0)

    def seq(j, s):
        wait_in(j, s)
        fire_in(j + 1, 1 - s)
        drain_outs(j - 1, 1 - s)
        rearrange(j, s)
        fire_outs(j, s)

    def body(jj, carry):
        j0 = jj * 2
        seq(j0, 0)
        seq(j0 + 1, 1)
        return carry

    lax.fori_loop(0, (NJ + 1) // 2, body, 0)

    @pl.when(wid < 4)
    def _():
        r0 = pl.multiple_of(wid * 8, 8)
        for t in range(W2 // 128):
            c0 = pl.multiple_of(TAILC + t * 128, 128)
            pltpu.sync_copy(tab_hbm.at[pl.ds(r0, 8), pl.ds(c0, 128)],
                            blk2.at[:, pl.ds(t * 128, 128)])

        def inner2(g, carry):
            off = g * 16
            for r in range(8):
                rowb2[pl.ds(r * W2 + off, 16)] = blk2[r, pl.ds(off, 16)]
            return carry
        lax.fori_loop(0, W2 // 16, inner2, 0, unroll=4)
        for r in range(8):
            o = pl.multiple_of((wid * 8 + r) * VOC + TAILC, 8)
            pltpu.sync_copy(rowb2.at[pl.ds(r * W2, W2)], flat.at[pl.ds(o, W2)])

    @pl.when(wid < NSP)
    def _():
        src = pl.multiple_of(wid * 64, 8)
        o = pl.multiple_of(wid * VOC + RAGC, 8)
        pltpu.sync_copy(tail_hbm.at[pl.ds(src, 64)], rowb2.at[pl.ds(0, 64)])
        pltpu.sync_copy(rowb2.at[pl.ds(0, 64)], flat.at[pl.ds(o, 64)])


@functools.partial(
    pl.kernel,
    mesh=_mesh,
    out_type=jax.ShapeDtypeStruct((BT,), jnp.float32),
    scratch_types=[
        pltpu.VMEM((ND + NSP, BPW), jnp.float32),
        pltpu.VMEM((ND, 16), jnp.float32),
        pltpu.VMEM((NIDX,), jnp.int32),
        pltpu.VMEM((NIDX,), jnp.float32),
        pltpu.VMEM((BPW,), jnp.float32),
        pltpu.SemaphoreType.DMA,
    ],
)
def _sc_linear(xt_hbm, tabf, w_hbm, out_hbm, xv, wv, idxv, gath, outv, sem):
    wid = lax.axis_index("s") * NCORE + lax.axis_index("c")
    base = wid * BPW

    pltpu.sync_copy(xt_hbm.at[:, pl.ds(base, BPW)], xv)
    pltpu.sync_copy(w_hbm, wv)
    wspl = [wv[d, :] for d in range(ND)]

    def build(c, carry):
        off = c * 16
        acc = jnp.zeros((16,), jnp.float32)
        for d in range(ND):
            acc = acc + xv[d, pl.ds(off, 16)] * wspl[d]
        outv[pl.ds(off, 16)] = acc
        for f in range(NSP):
            fv = xv[ND + f, pl.ds(off, 16)]
            idxv[pl.ds(f * BPW + off, 16)] = fv.astype(jnp.int32) + f * VOC
        return carry

    lax.fori_loop(0, NCHUNK, build, 0)

    def fire(j, carry):
        pltpu.make_async_copy(
            tabf.at[idxv.at[pl.ds(j * GCH, GCH)]],
            gath.at[pl.ds(j * GCH, GCH)],
            sem,
        ).start()
        return carry

    lax.fori_loop(0, NDMA, fire, 0)

    def drain(j, carry):
        pltpu.make_async_copy(
            tabf.at[idxv.at[pl.ds(j * GCH, GCH)]],
            gath.at[pl.ds(j * GCH, GCH)],
            sem,
        ).wait()
        return carry

    lax.fori_loop(0, NDMA, drain, 0)

    def reduce(c, carry):
        off = c * 16
        acc = outv[pl.ds(off, 16)]
        for f in range(NSP):
            acc = acc + gath[pl.ds(f * BPW + off, 16)]
        outv[pl.ds(off, 16)] = acc
        return carry

    lax.fori_loop(0, NCHUNK, reduce, 0)

    pltpu.sync_copy(outv, out_hbm.at[pl.ds(base, BPW)])


def kernel(X, tables, dense_w):
    xt = X.T
    w_rep = jnp.broadcast_to(dense_w.reshape(ND, 1), (ND, 16))
    tail = tables[:, RAGC:].reshape(-1)
    flat = _stage(tables, tail)
    out = _sc_linear(xt, flat, w_rep)
    idx = X[:, ND:].astype(jnp.int32)
    emb = tables[jnp.arange(NSP)[None, :], idx]
    sp = jnp.sum(emb, axis=-1)
    dense = jnp.dot(X[:, :ND], dense_w,
                    precision=jax.lax.Precision.HIGHEST)[:, 0]
    mine = sp + dense
    return (mine + 1e4 * (out - mine)).reshape(BT, 1)


# speedup vs baseline: 1.0955x; 1.0955x over previous
import functools

import jax
import jax.numpy as jnp
from jax import lax
from jax.experimental import pallas as pl
from jax.experimental.pallas import tpu as pltpu
from jax.experimental.pallas import tpu_sc as plsc

ND = 13
NSP = 26
VOC = 1000000
BT = 16384
FLAT = 32000000

_info = plsc.get_sparse_core_info()
NCORE = _info.num_cores
NSUB = _info.num_subcores
NW = NCORE * NSUB
BPW = BT // NW
NCHUNK = BPW // 16
NIDX = NSP * BPW
GCH = 128
NDMA = NIDX // GCH

W = 2048
NFULL = 488
TAILC = NFULL * W
W2 = 512
RAGC = TAILC + W2
NUNIT = NFULL * 4
NJ = (NUNIT + NW - 1) // NW

_mesh = plsc.VectorSubcoreMesh(core_axis_name="c", subcore_axis_name="s")


@functools.partial(
    pl.kernel,
    mesh=_mesh,
    out_type=jax.ShapeDtypeStruct((FLAT,), jnp.float32),
    scratch_types=[
        pltpu.VMEM((2, 8, W), jnp.float32),
        pltpu.VMEM((8 * W,), jnp.float32),
        pltpu.VMEM((8 * W,), jnp.float32),
        pltpu.VMEM((8, W2), jnp.float32),
        pltpu.VMEM((8 * W2,), jnp.float32),
        pltpu.SemaphoreType.DMA,
        pltpu.SemaphoreType.DMA,
        pltpu.SemaphoreType.DMA,
    ],
)
def _stage(tab_hbm, tail_hbm, flat, blk, rowb0, rowb1, blk2, rowb2,
           insem0, insem1, outsem):
    wid = lax.axis_index("s") * NCORE + lax.axis_index("c")
    insems = (insem0, insem1)
    rowbs = (rowb0, rowb1)

    def unit(j):
        u = wid + NW * j
        rb = u // NFULL
        c0 = (u % NFULL) * W
        r0 = pl.multiple_of(rb * 8, 8)
        c0 = pl.multiple_of(c0, 128)
        ok = jnp.logical_and(u >= 0, u < NUNIT)
        return ok, rb, r0, c0

    def in_copy(j, slot, t):
        _, rb, r0, c0 = unit(j)
        ct = pl.multiple_of(c0 + t * 128, 128)
        return pltpu.make_async_copy(
            tab_hbm.at[pl.ds(r0, 8), pl.ds(ct, 128)],
            blk.at[slot, :, pl.ds(t * 128, 128)], insems[slot],
        )

    def fire_in(j, slot):
        ok = unit(j)[0]

        def _go():
            for t in range(W // 128):
                in_copy(j, slot, t).start()
        pl.when(ok)(_go)

    def wait_in(j, slot):
        ok = unit(j)[0]

        def _go():
            for t in range(W // 128):
                in_copy(j, slot, t).wait()
        pl.when(ok)(_go)

    def rearrange(j, slot):
        ok = unit(j)[0]

        def _go():
            def inner(g, carry):
                off = g * 16
                for r in range(8):
                    v = blk[slot, r, pl.ds(off, 16)]
                    rowbs[slot][pl.ds(r * W + off, 16)] = v
                return carry
            lax.fori_loop(0, W // 16, inner, 0, unroll=4)
        pl.when(ok)(_go)

    def out_copy(j, slot, r):
        _, rb, r0, c0 = unit(j)
        o = pl.multiple_of((rb * 8 + r) * VOC + c0, 8)
        return pltpu.make_async_copy(
            rowbs[slot].at[pl.ds(r * W, W)], flat.at[pl.ds(o, W)], outsem,
        )

    def fire_outs(j, slot):
        ok = unit(j)[0]

        def _go():
            for r in range(8):
                out_copy(j, slot, r).start()
        pl.when(ok)(_go)

    def drain_outs(j, slot):
        ok = unit(j)[0]

        def _go():
            for r in range(8):
                out_copy(j, slot, r).wait()
        pl.when(ok)(_go)

    fire_in(0, 0)

    def seq(j, s):
        wait_in(j, s)
        fire_in(j + 1, 1 - s)
        drain_outs(j - 1, 1 - s)
        rearrange(j, s)
        fire_outs(j, s)

    def body(jj, carry):
        j0 = jj * 2
        seq(j0, 0)
        seq(j0 + 1, 1)
        return carry

    lax.fori_loop(0, (NJ + 1) // 2, body, 0)

    @pl.when(wid < 4)
    def _():
        r0 = pl.multiple_of(wid * 8, 8)
        for t in range(W2 // 128):
            c0 = pl.multiple_of(TAILC + t * 128, 128)
            pltpu.sync_copy(tab_hbm.at[pl.ds(r0, 8), pl.ds(c0, 128)],
                            blk2.at[:, pl.ds(t * 128, 128)])

        def inner2(g, carry):
            off = g * 16
            for r in range(8):
                rowb2[pl.ds(r * W2 + off, 16)] = blk2[r, pl.ds(off, 16)]
            return carry
        lax.fori_loop(0, W2 // 16, inner2, 0, unroll=4)
        for r in range(8):
            o = pl.multiple_of((wid * 8 + r) * VOC + TAILC, 8)
            pltpu.sync_copy(rowb2.at[pl.ds(r * W2, W2)], flat.at[pl.ds(o, W2)])

    @pl.when(wid < NSP)
    def _():
        src = pl.multiple_of(wid * 64, 8)
        o = pl.multiple_of(wid * VOC + RAGC, 8)
        pltpu.sync_copy(tail_hbm.at[pl.ds(src, 64)], rowb2.at[pl.ds(0, 64)])
        pltpu.sync_copy(rowb2.at[pl.ds(0, 64)], flat.at[pl.ds(o, 64)])


@functools.partial(
    pl.kernel,
    mesh=_mesh,
    out_type=jax.ShapeDtypeStruct((BT,), jnp.float32),
    scratch_types=[
        pltpu.VMEM((ND + NSP, BPW), jnp.float32),
        pltpu.VMEM((ND, 16), jnp.float32),
        pltpu.VMEM((NIDX,), jnp.int32),
        pltpu.VMEM((NIDX,), jnp.float32),
        pltpu.VMEM((BPW,), jnp.float32),
        pltpu.SemaphoreType.DMA,
    ],
)
def _sc_linear(xt_hbm, tabf, w_hbm, out_hbm, xv, wv, idxv, gath, outv, sem):
    wid = lax.axis_index("s") * NCORE + lax.axis_index("c")
    base = wid * BPW

    pltpu.sync_copy(xt_hbm.at[:, pl.ds(base, BPW)], xv)
    pltpu.sync_copy(w_hbm, wv)
    wspl = [wv[d, :] for d in range(ND)]

    def build(c, carry):
        off = c * 16
        acc = jnp.zeros((16,), jnp.float32)
        for d in range(ND):
            acc = acc + xv[d, pl.ds(off, 16)] * wspl[d]
        outv[pl.ds(off, 16)] = acc
        for f in range(NSP):
            fv = xv[ND + f, pl.ds(off, 16)]
            idxv[pl.ds(f * BPW + off, 16)] = fv.astype(jnp.int32) + f * VOC
        return carry

    lax.fori_loop(0, NCHUNK, build, 0)

    def fire(j, carry):
        pltpu.make_async_copy(
            tabf.at[idxv.at[pl.ds(j * GCH, GCH)]],
            gath.at[pl.ds(j * GCH, GCH)],
            sem,
        ).start()
        return carry

    lax.fori_loop(0, NDMA, fire, 0)

    def drain(j, carry):
        pltpu.make_async_copy(
            tabf.at[idxv.at[pl.ds(j * GCH, GCH)]],
            gath.at[pl.ds(j * GCH, GCH)],
            sem,
        ).wait()
        return carry

    lax.fori_loop(0, NDMA, drain, 0)

    def reduce(c, carry):
        off = c * 16
        acc = outv[pl.ds(off, 16)]
        for f in range(NSP):
            acc = acc + gath[pl.ds(f * BPW + off, 16)]
        outv[pl.ds(off, 16)] = acc
        return carry

    lax.fori_loop(0, NCHUNK, reduce, 0)

    pltpu.sync_copy(outv, out_hbm.at[pl.ds(base, BPW)])


def kernel(X, tables, dense_w):
    xt = X.T
    w_rep = jnp.broadcast_to(dense_w.reshape(ND, 1), (ND, 16))
    tail = tables[:, RAGC:].reshape(-1)
    flat = _stage(tables, tail)
    out = _sc_linear(xt, flat, w_rep)
    return out.reshape(BT, 1)


# speedup vs baseline: 1.0984x; 1.0027x over previous
import functools

import jax
import jax.numpy as jnp
from jax import lax
from jax.experimental import pallas as pl
from jax.experimental.pallas import tpu as pltpu
from jax.experimental.pallas import tpu_sc as plsc

ND = 13
NSP = 26
VOC = 1000000
BT = 16384
FLAT = 32000000

_info = plsc.get_sparse_core_info()
NCORE = _info.num_cores
NSUB = _info.num_subcores
NW = NCORE * NSUB
BPW = BT // NW
NCHUNK = BPW // 16
NIDX = NSP * BPW
GCH = 128
NDMA = NIDX // GCH

W = 2048
NFULL = 488
TAILC = NFULL * W
W2 = 512
RAGC = TAILC + W2
NUNIT = NFULL * 4
NJ = (NUNIT + NW - 1) // NW

_mesh = plsc.VectorSubcoreMesh(core_axis_name="c", subcore_axis_name="s")


@functools.partial(
    pl.kernel,
    mesh=_mesh,
    out_type=jax.ShapeDtypeStruct((FLAT,), jnp.float32),
    scratch_types=[
        pltpu.VMEM((2, 8, W), jnp.float32),
        pltpu.VMEM((8 * W,), jnp.float32),
        pltpu.VMEM((8 * W,), jnp.float32),
        pltpu.VMEM((8, W2), jnp.float32),
        pltpu.VMEM((8 * W2,), jnp.float32),
        pltpu.SemaphoreType.DMA,
        pltpu.SemaphoreType.DMA,
        pltpu.SemaphoreType.DMA,
    ],
)
def _stage(tab_hbm, tail_hbm, flat, blk, rowb0, rowb1, blk2, rowb2,
           insem0, insem1, outsem):
    wid = lax.axis_index("s") * NCORE + lax.axis_index("c")
    insems = (insem0, insem1)
    rowbs = (rowb0, rowb1)

    def unit(j):
        u = wid + NW * j
        rb = u // NFULL
        c0 = (u % NFULL) * W
        r0 = pl.multiple_of(rb * 8, 8)
        c0 = pl.multiple_of(c0, 128)
        ok = jnp.logical_and(u >= 0, u < NUNIT)
        return ok, rb, r0, c0

    def in_copy(j, slot, t):
        _, rb, r0, c0 = unit(j)
        ct = pl.multiple_of(c0 + t * 128, 128)
        return pltpu.make_async_copy(
            tab_hbm.at[pl.ds(r0, 8), pl.ds(ct, 128)],
            blk.at[slot, :, pl.ds(t * 128, 128)], insems[slot],
        )

    def fire_in(j, slot):
        ok = unit(j)[0]

        def _go():
            for t in range(W // 128):
                in_copy(j, slot, t).start()
        pl.when(ok)(_go)

    def wait_in(j, slot):
        ok = unit(j)[0]

        def _go():
            for t in range(W // 128):
                in_copy(j, slot, t).wait()
        pl.when(ok)(_go)

    def rearrange(j, slot):
        ok = unit(j)[0]

        def _go():
            def inner(g, carry):
                off = g * 16
                for r in range(8):
                    v = blk[slot, r, pl.ds(off, 16)]
                    rowbs[slot][pl.ds(r * W + off, 16)] = v
                return carry
            lax.fori_loop(0, W // 16, inner, 0, unroll=8)
        pl.when(ok)(_go)

    def out_copy(j, slot, r):
        _, rb, r0, c0 = unit(j)
        o = pl.multiple_of((rb * 8 + r) * VOC + c0, 8)
        return pltpu.make_async_copy(
            rowbs[slot].at[pl.ds(r * W, W)], flat.at[pl.ds(o, W)], outsem,
        )

    def fire_outs(j, slot):
        ok = unit(j)[0]

        def _go():
            for r in range(8):
                out_copy(j, slot, r).start()
        pl.when(ok)(_go)

    def drain_outs(j, slot):
        ok = unit(j)[0]

        def _go():
            for r in range(8):
                out_copy(j, slot, r).wait()
        pl.when(ok)(_go)

    fire_in(0, 0)

    def seq(j, s):
        wait_in(j, s)
        fire_in(j + 1, 1 - s)
        drain_outs(j - 1, 1 - s)
        rearrange(j, s)
        fire_outs(j, s)

    def body(jj, carry):
        j0 = jj * 2
        seq(j0, 0)
        seq(j0 + 1, 1)
        return carry

    lax.fori_loop(0, (NJ + 1) // 2, body, 0)

    @pl.when(wid < 4)
    def _():
        r0 = pl.multiple_of(wid * 8, 8)
        for t in range(W2 // 128):
            c0 = pl.multiple_of(TAILC + t * 128, 128)
            pltpu.sync_copy(tab_hbm.at[pl.ds(r0, 8), pl.ds(c0, 128)],
                            blk2.at[:, pl.ds(t * 128, 128)])

        def inner2(g, carry):
            off = g * 16
            for r in range(8):
                rowb2[pl.ds(r * W2 + off, 16)] = blk2[r, pl.ds(off, 16)]
            return carry
        lax.fori_loop(0, W2 // 16, inner2, 0, unroll=4)
        for r in range(8):
            o = pl.multiple_of((wid * 8 + r) * VOC + TAILC, 8)
            pltpu.sync_copy(rowb2.at[pl.ds(r * W2, W2)], flat.at[pl.ds(o, W2)])

    @pl.when(wid < NSP)
    def _():
        src = pl.multiple_of(wid * 64, 8)
        o = pl.multiple_of(wid * VOC + RAGC, 8)
        pltpu.sync_copy(tail_hbm.at[pl.ds(src, 64)], rowb2.at[pl.ds(0, 64)])
        pltpu.sync_copy(rowb2.at[pl.ds(0, 64)], flat.at[pl.ds(o, 64)])


@functools.partial(
    pl.kernel,
    mesh=_mesh,
    out_type=jax.ShapeDtypeStruct((BT,), jnp.float32),
    scratch_types=[
        pltpu.VMEM((ND + NSP, BPW), jnp.float32),
        pltpu.VMEM((ND, 16), jnp.float32),
        pltpu.VMEM((NIDX,), jnp.int32),
        pltpu.VMEM((NIDX,), jnp.float32),
        pltpu.VMEM((BPW,), jnp.float32),
        pltpu.SemaphoreType.DMA,
    ],
)
def _sc_linear(xt_hbm, tabf, w_hbm, out_hbm, xv, wv, idxv, gath, outv, sem):
    wid = lax.axis_index("s") * NCORE + lax.axis_index("c")
    base = wid * BPW

    pltpu.sync_copy(xt_hbm.at[:, pl.ds(base, BPW)], xv)
    pltpu.sync_copy(w_hbm, wv)
    wspl = [wv[d, :] for d in range(ND)]

    def build(c, carry):
        off = c * 16
        acc = jnp.zeros((16,), jnp.float32)
        for d in range(ND):
            acc = acc + xv[d, pl.ds(off, 16)] * wspl[d]
        outv[pl.ds(off, 16)] = acc
        for f in range(NSP):
            fv = xv[ND + f, pl.ds(off, 16)]
            idxv[pl.ds(f * BPW + off, 16)] = fv.astype(jnp.int32) + f * VOC
        return carry

    lax.fori_loop(0, NCHUNK, build, 0)

    def fire(j, carry):
        pltpu.make_async_copy(
            tabf.at[idxv.at[pl.ds(j * GCH, GCH)]],
            gath.at[pl.ds(j * GCH, GCH)],
            sem,
        ).start()
        return carry

    lax.fori_loop(0, NDMA, fire, 0)

    def drain(j, carry):
        pltpu.make_async_copy(
            tabf.at[idxv.at[pl.ds(j * GCH, GCH)]],
            gath.at[pl.ds(j * GCH, GCH)],
            sem,
        ).wait()
        return carry

    lax.fori_loop(0, NDMA, drain, 0)

    def reduce(c, carry):
        off = c * 16
        acc = outv[pl.ds(off, 16)]
        for f in range(NSP):
            acc = acc + gath[pl.ds(f * BPW + off, 16)]
        outv[pl.ds(off, 16)] = acc
        return carry

    lax.fori_loop(0, NCHUNK, reduce, 0)

    pltpu.sync_copy(outv, out_hbm.at[pl.ds(base, BPW)])


def kernel(X, tables, dense_w):
    xt = X.T
    w_rep = jnp.broadcast_to(dense_w.reshape(ND, 1), (ND, 16))
    tail = tables[:, RAGC:].reshape(-1)
    flat = _stage(tables, tail)
    out = _sc_linear(xt, flat, w_rep)
    return out.reshape(BT, 1)


# speedup vs baseline: 2.1620x; 1.9683x over previous
import functools

import jax
import jax.numpy as jnp
from jax import lax
from jax.experimental import pallas as pl
from jax.experimental.pallas import tpu as pltpu
from jax.experimental.pallas import tpu_sc as plsc

ND = 13
NSP = 26
VOC = 1000000
BT = 16384
FLAT = 32000000

_info = plsc.get_sparse_core_info()
NCORE = _info.num_cores
NSUB = _info.num_subcores
NW = NCORE * NSUB
BPW = BT // NW
NCHUNK = BPW // 16
NIDX = NSP * BPW
GCH = 128
NDMA = NIDX // GCH

W = 2048
NFULL = 488
TAILC = NFULL * W
W2 = 512
RAGC = TAILC + W2
NUNIT = NFULL * 4
NJ = (NUNIT + NW - 1) // NW

_mesh = plsc.VectorSubcoreMesh(core_axis_name="c", subcore_axis_name="s")


@functools.partial(
    pl.kernel,
    mesh=_mesh,
    out_type=jax.ShapeDtypeStruct((FLAT,), jnp.float32),
    scratch_types=[
        pltpu.VMEM((2, 8, W), jnp.float32),
        pltpu.VMEM((8 * W,), jnp.float32),
        pltpu.VMEM((8 * W,), jnp.float32),
        pltpu.VMEM((8, W2), jnp.float32),
        pltpu.VMEM((8 * W2,), jnp.float32),
        pltpu.SemaphoreType.DMA,
        pltpu.SemaphoreType.DMA,
        pltpu.SemaphoreType.DMA,
    ],
)
def _stage(tab_hbm, tail_hbm, flat, blk, rowb0, rowb1, blk2, rowb2,
           insem0, insem1, outsem):
    wid = lax.axis_index("s") * NCORE + lax.axis_index("c")
    insems = (insem0, insem1)
    rowbs = (rowb0, rowb1)

    def unit(j):
        u = wid + NW * j
        rb = u // NFULL
        c0 = (u % NFULL) * W
        r0 = pl.multiple_of(rb * 8, 8)
        c0 = pl.multiple_of(c0, 128)
        ok = jnp.logical_and(u >= 0, u < NUNIT)
        return ok, rb, r0, c0

    def in_copy(j, slot, t):
        _, rb, r0, c0 = unit(j)
        ct = pl.multiple_of(c0 + t * 128, 128)
        return pltpu.make_async_copy(
            tab_hbm.at[pl.ds(r0, 8), pl.ds(ct, 128)],
            blk.at[slot, :, pl.ds(t * 128, 128)], insems[slot],
        )

    def fire_in(j, slot):
        ok = unit(j)[0]

        def _go():
            for t in range(W // 128):
                in_copy(j, slot, t).start()
        pl.when(ok)(_go)

    def wait_in(j, slot):
        ok = unit(j)[0]

        def _go():
            for t in range(W // 128):
                in_copy(j, slot, t).wait()
        pl.when(ok)(_go)

    def rearrange(j, slot):
        ok = unit(j)[0]

        def _go():
            @plsc.parallel_loop(0, W // 16, unroll=8)
            def inner(g):
                off = g * 16
                for r in range(8):
                    v = blk[slot, r, pl.ds(off, 16)]
                    rowbs[slot][pl.ds(r * W + off, 16)] = v
        pl.when(ok)(_go)

    def out_copy(j, slot, r):
        _, rb, r0, c0 = unit(j)
        o = pl.multiple_of((rb * 8 + r) * VOC + c0, 8)
        return pltpu.make_async_copy(
            rowbs[slot].at[pl.ds(r * W, W)], flat.at[pl.ds(o, W)], outsem,
        )

    def fire_outs(j, slot):
        ok = unit(j)[0]

        def _go():
            for r in range(8):
                out_copy(j, slot, r).start()
        pl.when(ok)(_go)

    def drain_outs(j, slot):
        ok = unit(j)[0]

        def _go():
            for r in range(8):
                out_copy(j, slot, r).wait()
        pl.when(ok)(_go)

    fire_in(0, 0)

    def seq(j, s):
        wait_in(j, s)
        fire_in(j + 1, 1 - s)
        drain_outs(j - 1, 1 - s)
        rearrange(j, s)
        fire_outs(j, s)

    def body(jj, carry):
        j0 = jj * 2
        seq(j0, 0)
        seq(j0 + 1, 1)
        return carry

    lax.fori_loop(0, (NJ + 1) // 2, body, 0)

    @pl.when(wid < 4)
    def _():
        r0 = pl.multiple_of(wid * 8, 8)
        for t in range(W2 // 128):
            c0 = pl.multiple_of(TAILC + t * 128, 128)
            pltpu.sync_copy(tab_hbm.at[pl.ds(r0, 8), pl.ds(c0, 128)],
                            blk2.at[:, pl.ds(t * 128, 128)])

        @plsc.parallel_loop(0, W2 // 16, unroll=4)
        def inner2(g):
            off = g * 16
            for r in range(8):
                rowb2[pl.ds(r * W2 + off, 16)] = blk2[r, pl.ds(off, 16)]
        for r in range(8):
            o = pl.multiple_of((wid * 8 + r) * VOC + TAILC, 8)
            pltpu.sync_copy(rowb2.at[pl.ds(r * W2, W2)], flat.at[pl.ds(o, W2)])

    @pl.when(wid < NSP)
    def _():
        src = pl.multiple_of(wid * 64, 8)
        o = pl.multiple_of(wid * VOC + RAGC, 8)
        pltpu.sync_copy(tail_hbm.at[pl.ds(src, 64)], rowb2.at[pl.ds(0, 64)])
        pltpu.sync_copy(rowb2.at[pl.ds(0, 64)], flat.at[pl.ds(o, 64)])


@functools.partial(
    pl.kernel,
    mesh=_mesh,
    out_type=jax.ShapeDtypeStruct((BT,), jnp.float32),
    scratch_types=[
        pltpu.VMEM((ND + NSP, BPW), jnp.float32),
        pltpu.VMEM((ND, 16), jnp.float32),
        pltpu.VMEM((NIDX,), jnp.int32),
        pltpu.VMEM((NIDX,), jnp.float32),
        pltpu.VMEM((BPW,), jnp.float32),
        pltpu.SemaphoreType.DMA,
    ],
)
def _sc_linear(xt_hbm, tabf, w_hbm, out_hbm, xv, wv, idxv, gath, outv, sem):
    wid = lax.axis_index("s") * NCORE + lax.axis_index("c")
    base = wid * BPW

    pltpu.sync_copy(xt_hbm.at[:, pl.ds(base, BPW)], xv)
    pltpu.sync_copy(w_hbm, wv)
    wspl = [wv[d, :] for d in range(ND)]

    def build(c, carry):
        off = c * 16
        acc = jnp.zeros((16,), jnp.float32)
        for d in range(ND):
            acc = acc + xv[d, pl.ds(off, 16)] * wspl[d]
        outv[pl.ds(off, 16)] = acc
        for f in range(NSP):
            fv = xv[ND + f, pl.ds(off, 16)]
            idxv[pl.ds(f * BPW + off, 16)] = fv.astype(jnp.int32) + f * VOC
        return carry

    lax.fori_loop(0, NCHUNK, build, 0)

    def fire(j, carry):
        pltpu.make_async_copy(
            tabf.at[idxv.at[pl.ds(j * GCH, GCH)]],
            gath.at[pl.ds(j * GCH, GCH)],
            sem,
        ).start()
        return carry

    lax.fori_loop(0, NDMA, fire, 0)

    def drain(j, carry):
        pltpu.make_async_copy(
            tabf.at[idxv.at[pl.ds(j * GCH, GCH)]],
            gath.at[pl.ds(j * GCH, GCH)],
            sem,
        ).wait()
        return carry

    lax.fori_loop(0, NDMA, drain, 0)

    def reduce(c, carry):
        off = c * 16
        acc = outv[pl.ds(off, 16)]
        for f in range(NSP):
            acc = acc + gath[pl.ds(f * BPW + off, 16)]
        outv[pl.ds(off, 16)] = acc
        return carry

    lax.fori_loop(0, NCHUNK, reduce, 0)

    pltpu.sync_copy(outv, out_hbm.at[pl.ds(base, BPW)])


def kernel(X, tables, dense_w):
    xt = X.T
    w_rep = jnp.broadcast_to(dense_w.reshape(ND, 1), (ND, 16))
    tail = tables[:, RAGC:].reshape(-1)
    flat = _stage(tables, tail)
    out = _sc_linear(xt, flat, w_rep)
    return out.reshape(BT, 1)
